# Initial kernel scaffold; baseline (speedup 1.0000x reference)
import jax
import jax.numpy as jnp
from jax.experimental import pallas as pl


def kernel(pred_logits, pred_keypoints, tgt_ids, tgt_keypoints, num_boxes):
    raise NotImplementedError("write your pallas kernel here")


# speedup vs baseline: 1.0211x; 1.0211x over previous
import jax
import jax.numpy as jnp
from jax.experimental import pallas as pl
from jax.experimental.pallas import tpu as pltpu

_NBLK = 400


def _cost_body(logits_ref, kp_ref, tgtT_ref, oh_ref, invnb_ref, out_ref):
    inv = invnb_ref[0, 0]
    nblk = logits_ref.shape[0]
    t = tgtT_ref.shape[1]
    nc = logits_ref.shape[1]

    lg = logits_ref[...]
    m = jnp.max(lg, axis=1, keepdims=True)
    e = jnp.exp(lg - m)
    p = e / jnp.sum(e, axis=1, keepdims=True)
    cls = p[:, 0:1] * oh_ref[0:1, :]
    for c in range(1, nc):
        cls = cls + p[:, c : c + 1] * oh_ref[c : c + 1, :]

    s_vis = jnp.sqrt(0.2 * inv)
    s_ctr = jnp.sqrt(0.5 * inv)
    vp = kp_ref[:, 36:53] * s_vis
    vg = tgtT_ref[36:53, :] * s_vis
    cp2 = kp_ref[:, 0:2] * s_ctr
    cg2 = tgtT_ref[0:2, :] * s_ctr

    acc = jnp.zeros((nblk, t), jnp.float32) - cls
    for d in range(17):
        diff = vp[:, d : d + 1] - vg[d : d + 1, :]
        acc = acc + diff * diff
    for d in range(2):
        diff = cp2[:, d : d + 1] - cg2[d : d + 1, :]
        acc = acc + diff * diff

    a = 0.5 * inv
    zp = kp_ref[:, 2:36] * a
    zg = tgtT_ref[2:36, :] * a
    cp = kp_ref[:, 0:2] * a
    cg = tgtT_ref[0:2, :] * a
    visg = tgtT_ref[36:53, :]
    dcs = (cp[:, 0:1] - cg[0:1, :], cp[:, 1:2] - cg[1:2, :])
    for d in range(34):
        dz = zp[:, d : d + 1] - zg[d : d + 1, :]
        term = jnp.abs(dz) + 8.0 * jnp.abs(dz + dcs[d % 2])
        acc = acc + term * visg[d // 2 : d // 2 + 1, :]

    out_ref[...] = acc


def kernel(pred_logits, pred_keypoints, tgt_ids, tgt_keypoints, num_boxes):
    bs, q, nc = pred_logits.shape
    n = bs * q
    t = tgt_keypoints.shape[0]

    logits2d = pred_logits.reshape(n, nc)
    kp2d = pred_keypoints.reshape(n, 53)
    tgt_t = tgt_keypoints.T
    onehot = (tgt_ids[None, :] == jnp.arange(nc)[:, None]).astype(jnp.float32)
    invnb = (1.0 / jnp.asarray(num_boxes, jnp.float32)).reshape(1, 1)

    grid = (n // _NBLK,)
    out = pl.pallas_call(
        _cost_body,
        grid=grid,
        in_specs=[
            pl.BlockSpec((_NBLK, nc), lambda i: (i, 0)),
            pl.BlockSpec((_NBLK, 53), lambda i: (i, 0)),
            pl.BlockSpec((53, t), lambda i: (0, 0)),
            pl.BlockSpec((nc, t), lambda i: (0, 0)),
            pl.BlockSpec((1, 1), lambda i: (0, 0)),
        ],
        out_specs=pl.BlockSpec((_NBLK, t), lambda i: (i, 0)),
        out_shape=jax.ShapeDtypeStruct((n, t), jnp.float32),
        compiler_params=pltpu.CompilerParams(
            dimension_semantics=("parallel",),
        ),
    )(logits2d, kp2d, tgt_t, onehot, invnb)
    return out.reshape(bs, q, t)


# speedup vs baseline: 1.1603x; 1.1363x over previous
import jax
import jax.numpy as jnp
from jax.experimental import pallas as pl
from jax.experimental.pallas import tpu as pltpu

_NBLK = 400
_K = 128


def _cost_body(logits_ref, kp_ref, tgtT_ref, oh_ref, invnb_ref, out_ref,
               a_ref, b_ref):
    inv = invnb_ref[0, 0]
    nblk = logits_ref.shape[0]
    nc = logits_ref.shape[1]

    lg = logits_ref[...]
    m = jnp.max(lg, axis=1, keepdims=True)
    e = jnp.exp(lg - m)
    p = e / jnp.sum(e, axis=1, keepdims=True)

    w_vis = 0.2 * inv
    w_ctr = 0.5 * inv
    vp = kp_ref[:, 36:53]
    cp = kp_ref[:, 0:2]
    vg = tgtT_ref[36:53, :]
    cg = tgtT_ref[0:2, :]

    a_ref[...] = jnp.zeros(a_ref.shape, jnp.float32)
    a_ref[:, 0:17] = (-2.0 * w_vis) * vp
    a_ref[:, 17:19] = (-2.0 * w_ctr) * cp
    a_ref[:, 19:20] = (w_vis * jnp.sum(vp * vp, axis=1, keepdims=True)
                       + w_ctr * jnp.sum(cp * cp, axis=1, keepdims=True))
    a_ref[:, 20:21] = jnp.ones((nblk, 1), jnp.float32)
    a_ref[:, 21:21 + nc] = -p

    b_ref[...] = jnp.zeros(b_ref.shape, jnp.float32)
    b_ref[0:17, :] = vg
    b_ref[17:19, :] = cg
    b_ref[19:20, :] = jnp.ones((1, b_ref.shape[1]), jnp.float32)
    b_ref[20:21, :] = (w_vis * jnp.sum(vg * vg, axis=0, keepdims=True)
                       + w_ctr * jnp.sum(cg * cg, axis=0, keepdims=True))
    b_ref[21:21 + nc, :] = oh_ref[...]

    acc = jnp.dot(a_ref[...], b_ref[...],
                  preferred_element_type=jnp.float32)

    a = 0.5 * inv
    zp = kp_ref[:, 2:36] * a
    zg = tgtT_ref[2:36, :] * a
    cps = cp * a
    cgs = cg * a
    dcx = cps[:, 0:1] - cgs[0:1, :]
    dcy = cps[:, 1:2] - cgs[1:2, :]
    for k in range(17):
        d0, d1 = 2 * k, 2 * k + 1
        dz0 = zp[:, d0 : d0 + 1] - zg[d0 : d0 + 1, :]
        dz1 = zp[:, d1 : d1 + 1] - zg[d1 : d1 + 1, :]
        s1 = jnp.abs(dz0) + jnp.abs(dz1)
        s2 = jnp.abs(dz0 + dcx) + jnp.abs(dz1 + dcy)
        acc = acc + (s1 + 8.0 * s2) * vg[k : k + 1, :]
    out_ref[...] = acc


def kernel(pred_logits, pred_keypoints, tgt_ids, tgt_keypoints, num_boxes):
    bs, q, nc = pred_logits.shape
    n = bs * q
    t = tgt_keypoints.shape[0]

    logits2d = pred_logits.reshape(n, nc)
    kp2d = pred_keypoints.reshape(n, 53)
    tgt_t = tgt_keypoints.T
    onehot = (tgt_ids[None, :] == jnp.arange(nc)[:, None]).astype(jnp.float32)
    invnb = (1.0 / jnp.asarray(num_boxes, jnp.float32)).reshape(1, 1)

    grid = (n // _NBLK,)
    out = pl.pallas_call(
        _cost_body,
        grid=grid,
        in_specs=[
            pl.BlockSpec((_NBLK, nc), lambda i: (i, 0)),
            pl.BlockSpec((_NBLK, 53), lambda i: (i, 0)),
            pl.BlockSpec((53, t), lambda i: (0, 0)),
            pl.BlockSpec((nc, t), lambda i: (0, 0)),
            pl.BlockSpec((1, 1), lambda i: (0, 0)),
        ],
        out_specs=pl.BlockSpec((_NBLK, t), lambda i: (i, 0)),
        out_shape=jax.ShapeDtypeStruct((n, t), jnp.float32),
        scratch_shapes=[
            pltpu.VMEM((_NBLK, _K), jnp.float32),
            pltpu.VMEM((_K, t), jnp.float32),
        ],
        compiler_params=pltpu.CompilerParams(
            dimension_semantics=("parallel",),
        ),
    )(logits2d, kp2d, tgt_t, onehot, invnb)
    return out.reshape(bs, q, t)
